# baseline (speedup 1.0000x reference)
import functools

import jax
import jax.numpy as jnp
from jax import lax
from jax.experimental import pallas as pl
from jax.experimental.pallas import tpu as pltpu
from jax.experimental.pallas import tpu_sc as plsc

N = 10000
NPAD = 10048
E = 320000
HIDDEN = 64
NCLUST = 1000
NW = 32
PAIR = 314
NR = 157
TILE_E = 2000
LOC_CAP = 4096
FLUSH = 2048
DUMMY = 511 * 16384
SLAB_W = E + FLUSH
ROW_BLK = 1256

_SC_PARAMS = pltpu.CompilerParams(needs_layout_passes=False)


def _mesh():
    return plsc.VectorSubcoreMesh(core_axis_name="c", subcore_axis_name="s")


def _wid():
    return lax.axis_index("s") * 2 + lax.axis_index("c")


def _mlp_body(xa_ref, xb_ref, w1a_ref, w1b_ref, b1_ref, g_ref, be_ref,
              w2_ref, b2_ref, o_ref, o2_ref):
    h = jnp.dot(xa_ref[...], w1a_ref[...], preferred_element_type=jnp.float32)
    h = h + jnp.dot(xb_ref[...], w1b_ref[...], preferred_element_type=jnp.float32)
    h = h + b1_ref[...]
    mu = jnp.mean(h, axis=-1, keepdims=True)
    var = jnp.mean((h - mu) * (h - mu), axis=-1, keepdims=True)
    h = (h - mu) * lax.rsqrt(var + 1e-5) * g_ref[...] + be_ref[...]
    h = jnp.maximum(h, 0.0)
    x2 = jnp.dot(h, w2_ref[...], preferred_element_type=jnp.float32) + b2_ref[...]
    o_ref[...] = x2
    o2_ref[...] = x2.astype(jnp.bfloat16)


def _mlp(xa, xb, W1, b1, g, be, W2, b2, perm):
    ca = xa.shape[1]
    if xb is None:
        xb = jnp.zeros((NPAD, 8), jnp.float32)
        w1b = jnp.zeros((8, HIDDEN), jnp.float32)
    else:
        w1b = W1[:, ca:].T
        if perm is not None:
            w1b = w1b[perm]
    w1a = W1[:, :ca].T
    cout = W2.shape[0]
    grid = NPAD // ROW_BLK
    return pl.pallas_call(
        _mlp_body,
        grid=(grid,),
        in_specs=[
            pl.BlockSpec((ROW_BLK, xa.shape[1]), lambda i: (i, 0)),
            pl.BlockSpec((ROW_BLK, xb.shape[1]), lambda i: (i, 0)),
            pl.BlockSpec((xa.shape[1], HIDDEN), lambda i: (0, 0)),
            pl.BlockSpec((xb.shape[1], HIDDEN), lambda i: (0, 0)),
            pl.BlockSpec((1, HIDDEN), lambda i: (0, 0)),
            pl.BlockSpec((1, HIDDEN), lambda i: (0, 0)),
            pl.BlockSpec((1, HIDDEN), lambda i: (0, 0)),
            pl.BlockSpec((HIDDEN, cout), lambda i: (0, 0)),
            pl.BlockSpec((1, cout), lambda i: (0, 0)),
        ],
        out_specs=[pl.BlockSpec((ROW_BLK, cout), lambda i: (i, 0)),
                   pl.BlockSpec((ROW_BLK, cout), lambda i: (i, 0))],
        out_shape=[jax.ShapeDtypeStruct((NPAD, cout), jnp.float32),
                   jax.ShapeDtypeStruct((NPAD, cout), jnp.bfloat16)],
    )(xa, xb, w1a, w1b, b1[None], g[None], be[None], W2.T, b2[None])


def _prep_body(src_hbm, dst_hbm, slab_hbm, counts_hbm, stile, dtile, loc, cntv):
    w = _wid()
    base = w * PAIR

    def init(i, _):
        loc[pl.ds(i * 16, 16)] = jnp.full((16,), DUMMY, jnp.int32)
        return 0
    lax.fori_loop(0, LOC_CAP // 16, init, 0)

    def tile_body(t, carry):
        off, written = carry
        pltpu.sync_copy(src_hbm.at[pl.ds(t * TILE_E, TILE_E)], stile)
        pltpu.sync_copy(dst_hbm.at[pl.ds(t * TILE_E, TILE_E)], dtile)

        def grp(g, off):
            d = dtile[pl.ds(g * 16, 16)]
            s = stile[pl.ds(g * 16, 16)]
            drel = d - base
            m = (drel >= 0) & (drel < PAIR)
            cs = plsc.cumsum(jnp.where(m, 1, 0))
            plsc.store_scatter(loc, [off + cs - 1], drel * 16384 + s, mask=m)
            return off + cs[15]
        off = lax.fori_loop(0, TILE_E // 16, grp, off)

        def do_flush(c):
            off, written = c
            pltpu.sync_copy(loc.at[pl.ds(0, FLUSH)],
                            slab_hbm.at[pl.ds(pl.multiple_of(w * SLAB_W + written, 2048), FLUSH)])

            def shift(i, _):
                loc[pl.ds(i * 16, 16)] = loc[pl.ds(FLUSH + i * 16, 16)]
                return 0
            lax.fori_loop(0, FLUSH // 16, shift, 0)
            return off - FLUSH, written + FLUSH
        off, written = lax.cond(off >= FLUSH, do_flush, lambda c: c, (off, written))
        return off, written

    off, written = lax.fori_loop(0, E // TILE_E, tile_body, (0, 0))

    def final_flush(c):
        off, written = c
        pltpu.sync_copy(loc.at[pl.ds(0, FLUSH)],
                        slab_hbm.at[pl.ds(pl.multiple_of(w * SLAB_W + written, 2048), FLUSH)])
        return 0, written + FLUSH
    off, written = lax.cond(off > 0, final_flush, lambda c: c, (off, written))

    cntv[pl.ds(0, 16)] = jnp.full((16,), written, jnp.int32)
    pltpu.sync_copy(cntv, counts_hbm.at[pl.ds(pl.multiple_of(w * 16, 16), 16)])


def _edge_prep(src, dst):
    return pl.kernel(
        _prep_body,
        out_type=(jax.ShapeDtypeStruct((NW * SLAB_W,), jnp.int32),
                  jax.ShapeDtypeStruct((NW * 16,), jnp.int32)),
        mesh=_mesh(),
        compiler_params=_SC_PARAMS,
        scratch_types=[
            pltpu.VMEM((TILE_E,), jnp.int32),
            pltpu.VMEM((TILE_E,), jnp.int32),
            pltpu.VMEM((LOC_CAP,), jnp.int32),
            pltpu.VMEM((16,), jnp.int32),
        ],
    )(src, dst)


def _agg_body(c, B, packed, x2i_hbm, slab_hbm, counts_hbm, agg_hbm,
              ltile, ldst, lsrc, idxb0, idxb1, gbuf0, gbuf1, acc, cntv,
              sem0, sem1):
    w = _wid()

    pltpu.sync_copy(counts_hbm.at[pl.ds(pl.multiple_of(w * 16, 16), 16)], cntv)
    cnt = cntv[pl.ds(0, 16)][0]
    ntiles = cnt // FLUSH

    def init_z(i, _):
        ldst[pl.ds(i * 16, 16)] = jnp.zeros((16,), jnp.int32)
        lsrc[pl.ds(i * 16, 16)] = jnp.zeros((16,), jnp.int32)
        return 0
    lax.fori_loop(0, (FLUSH + 144) // 16, init_z, 0)

    lane = lax.iota(jnp.int32, 16)

    for p in range(2):
        r = 2 * w + p
        rbase = p * NR

        def init_acc(i, _):
            acc[pl.ds(i * 16, 16)] = jnp.full((16,), -jnp.inf, jnp.float32)
            return 0
        lax.fori_loop(0, NR * c // 16, init_acc, 0)

        def tile_body(t, _):
            pltpu.sync_copy(slab_hbm.at[pl.ds(pl.multiple_of(w * SLAB_W + t * FLUSH, 2048), FLUSH)], ltile)

            def grp(g, off):
                pk = ltile[pl.ds(g * 16, 16)]
                dl = lax.shift_right_logical(pk, 14) - rbase
                s = pk & 16383
                m = (dl >= 0) & (dl < NR)
                cs = plsc.cumsum(jnp.where(m, 1, 0))
                idx = off + cs - 1
                plsc.store_scatter(ldst, [idx], dl, mask=m)
                plsc.store_scatter(lsrc, [idx], s, mask=m)
                return off + cs[15]
            off = lax.fori_loop(0, FLUSH // 16, grp, 0)

            base16 = (off // 16) * 16
            for k in range(8):
                gs = base16 + 16 * k
                v = ldst[pl.ds(gs, 16)]
                ldst[pl.ds(gs, 16)] = jnp.where(gs + lane >= off, NR, v)
            nb = (off + B - 1) // B

            def fire(b, idxb, gbuf, sem):
                for u in range(B // 16):
                    idxb[pl.ds(u * 16, 16)] = lsrc[pl.ds(b * B + u * 16, 16)]
                pltpu.async_copy(x2i_hbm.at[idxb], gbuf, sem)

            def wait(idxb, gbuf, sem):
                pltpu.make_async_copy(x2i_hbm.at[idxb], gbuf, sem).wait()

            def process(b, gbuf):
                def pgrp(gg, _):
                    dv = ldst[pl.ds(b * B + gg * 16, 16)]
                    for l in range(16):
                        rowb = dv[l] * c
                        e = gg * 16 + l
                        if packed:
                            for u2 in range(c // 32):
                                wv = gbuf[e, pl.ds(u2 * 16, 16)]
                                lo = plsc.bitcast(lax.shift_left(wv, 16), jnp.float32)
                                hi = plsc.bitcast(wv & (-65536), jnp.float32)
                                cb = rowb + u2 * 32
                                acc[pl.ds(cb, 16)] = jnp.maximum(acc[pl.ds(cb, 16)], lo)
                                acc[pl.ds(cb + 16, 16)] = jnp.maximum(
                                    acc[pl.ds(cb + 16, 16)], hi)
                        else:
                            for u in range(c // 16):
                                acc[pl.ds(rowb + u * 16, 16)] = jnp.maximum(
                                    acc[pl.ds(rowb + u * 16, 16)],
                                    gbuf[e, pl.ds(u * 16, 16)])
                    return 0
                lax.fori_loop(0, B // 16, pgrp, 0)

            @pl.when(nb > 0)
            def _():
                fire(0, idxb0, gbuf0, sem0)

                def pair(b2, _):
                    b0 = 2 * b2
                    wait(idxb0, gbuf0, sem0)

                    @pl.when(b0 + 1 < nb)
                    def _():
                        fire(b0 + 1, idxb1, gbuf1, sem1)
                    process(b0, gbuf0)

                    @pl.when(b0 + 1 < nb)
                    def _():
                        wait(idxb1, gbuf1, sem1)

                        @pl.when(b0 + 2 < nb)
                        def _():
                            fire(b0 + 2, idxb0, gbuf0, sem0)
                        process(b0 + 1, gbuf1)
                    return 0
                lax.fori_loop(0, (nb + 1) // 2, pair, 0)
            return 0
        lax.fori_loop(0, ntiles, tile_body, 0)

        def fixup(i, _):
            v = acc[pl.ds(i * 16, 16)]
            acc[pl.ds(i * 16, 16)] = jnp.where(v < -1e37, 0.0, v)
            return 0
        lax.fori_loop(0, NR * c // 16, fixup, 0)
        pltpu.sync_copy(acc.at[pl.ds(0, NR * c)],
                        agg_hbm.at[pl.ds(pl.multiple_of(r * NR * c, 128), NR * c)])


def _edge_agg(x2i, slab, counts, nchunk):
    c = nchunk * 128
    packed = c > 128
    B = min(128, (32768 if packed else 16384) // c)
    wpr = c // 2 if packed else c
    out = pl.kernel(
        functools.partial(_agg_body, c, B, packed),
        out_type=jax.ShapeDtypeStruct((NPAD * c,), jnp.float32),
        mesh=_mesh(),
        compiler_params=_SC_PARAMS,
        scratch_types=[
            pltpu.VMEM((FLUSH,), jnp.int32),
            pltpu.VMEM((FLUSH + 144,), jnp.int32),
            pltpu.VMEM((FLUSH + 144,), jnp.int32),
            pltpu.VMEM((B,), jnp.int32),
            pltpu.VMEM((B,), jnp.int32),
            pltpu.VMEM((B, wpr), jnp.int32 if packed else jnp.float32),
            pltpu.VMEM((B, wpr), jnp.int32 if packed else jnp.float32),
            pltpu.VMEM(((NR + 1) * c,), jnp.float32),
            pltpu.VMEM((16,), jnp.int32),
            pltpu.SemaphoreType.DMA,
            pltpu.SemaphoreType.DMA,
        ],
    )(x2i, slab, counts)
    return out.reshape(NPAD, c)


CL_PER = 32
CTILE = 2512
RTILE = 64


def _pool_body(xa_hbm, xb_hbm, cl_hbm, out_hbm, ctile, crow, ra, rb, acc, sem):
    w = _wid()
    cbase = w * CL_PER

    def init_acc(i, _):
        for u in range(64):
            acc[i, pl.ds(u * 16, 16)] = jnp.full((16,), -jnp.inf, jnp.float32)
        return 0
    lax.fori_loop(0, CL_PER, init_acc, 0)

    def scan_tile(t, carry):
        lo, n = carry
        pltpu.sync_copy(cl_hbm.at[pl.ds(pl.multiple_of(t * CTILE, 16), CTILE)], ctile)

        def grp(g, carry):
            lo, n = carry
            v = ctile[pl.ds(g * 16, 16)]
            lo = lo + jnp.sum(jnp.where(v < cbase, 1, 0))
            n = n + jnp.sum(jnp.where((v >= cbase) & (v < cbase + CL_PER), 1, 0))
            return lo, n
        return lax.fori_loop(0, CTILE // 16, grp, carry)
    lo, nrows = lax.fori_loop(0, NPAD // CTILE, scan_tile, (0, 0))

    t0 = lo // RTILE
    t1 = (lo + nrows + RTILE - 1) // RTILE

    def tile_body(t, _):
        pltpu.sync_copy(xa_hbm.at[pl.ds(pl.multiple_of(t * RTILE, RTILE), RTILE), :], ra)
        pltpu.sync_copy(xb_hbm.at[pl.ds(pl.multiple_of(t * RTILE, RTILE), RTILE), :], rb)
        pltpu.sync_copy(cl_hbm.at[pl.ds(pl.multiple_of(t * RTILE, RTILE), RTILE)], crow.at[pl.ds(0, RTILE)])

        def row(e, _):
            d = crow[pl.ds(e, 16)][0] - cbase
            ok = (d >= 0) & (d < CL_PER)
            dc = jnp.clip(d, 0, CL_PER - 1)
            for u in range(32):
                va = jnp.where(ok, ra[e, pl.ds(u * 16, 16)], -jnp.inf)
                acc[dc, pl.ds(u * 16, 16)] = jnp.maximum(acc[dc, pl.ds(u * 16, 16)], va)
                vb = jnp.where(ok, rb[e, pl.ds(u * 16, 16)], -jnp.inf)
                acc[dc, pl.ds(512 + u * 16, 16)] = jnp.maximum(
                    acc[dc, pl.ds(512 + u * 16, 16)], vb)
            return 0
        lax.fori_loop(0, RTILE, row, 0)
        return 0
    lax.fori_loop(t0, t1, tile_body, 0)

    def fixup(i, _):
        for u in range(64):
            v = acc[i, pl.ds(u * 16, 16)]
            acc[i, pl.ds(u * 16, 16)] = jnp.where(v < -1e37, 0.0, v)
        return 0
    lax.fori_loop(0, CL_PER, fixup, 0)
    pltpu.sync_copy(acc, out_hbm.at[pl.ds(pl.multiple_of(w * CL_PER, CL_PER), CL_PER), :])


def _pool(xa, xb, cl_pad):
    return pl.kernel(
        _pool_body,
        out_type=jax.ShapeDtypeStruct((NW * CL_PER, 1024), jnp.float32),
        mesh=_mesh(),
        compiler_params=_SC_PARAMS,
        scratch_types=[
            pltpu.VMEM((CTILE,), jnp.int32),
            pltpu.VMEM((RTILE + 16,), jnp.int32),
            pltpu.VMEM((RTILE, 512), jnp.float32),
            pltpu.VMEM((RTILE, 512), jnp.float32),
            pltpu.VMEM((CL_PER, 1024), jnp.float32),
            pltpu.SemaphoreType.DMA,
        ],
    )(xa, xb, cl_pad)


def _norm_body(p_ref, o_ref):
    v = p_ref[...]
    ss = jnp.sum(v * v, axis=0, keepdims=True)
    o_ref[...] = v[:NCLUST, :] * lax.rsqrt(ss)


def _norm(pooled_pad):
    return pl.pallas_call(
        _norm_body,
        grid=(8,),
        in_specs=[pl.BlockSpec((NW * CL_PER, 128), lambda i: (0, i))],
        out_specs=pl.BlockSpec((NCLUST, 128), lambda i: (0, i)),
        out_shape=jax.ShapeDtypeStruct((NCLUST, 1024), jnp.float32),
    )(pooled_pad)


def _perm(c):
    import numpy as _np
    p = _np.arange(c).reshape(c // 32, 2, 16)
    return _np.concatenate([p[:, 0] * 1, p[:, 1]], axis=-1).reshape(-1) * 0 + _np.array(
        [b * 32 + 2 * k + h for b in range(c // 32) for h in range(2) for k in range(16)])


def kernel(x, edge_index, batch, cluster, W1_0, b1_0, g_0, be_0, W2_0, b2_0,
           W1_1, b1_1, g_1, be_1, W2_1, b2_1, W1_2, b1_2, g_2, be_2, W2_2, b2_2):
    import numpy as np
    src = edge_index[0]
    dst = edge_index[1]
    slab, counts = _edge_prep(src, dst)

    xa = jnp.pad(x, ((0, NPAD - N), (0, 0)))
    xb = None
    perm = None
    layers = [(W1_0, b1_0, g_0, be_0, W2_0, b2_0, 1),
              (W1_1, b1_1, g_1, be_1, W2_1, b2_1, 2),
              (W1_2, b1_2, g_2, be_2, W2_2, b2_2, 4)]
    for (W1, b1, g, be, W2, b2, nchunk) in layers:
        x2 = _mlp(xa, xb, W1, b1, g, be, W2, b2, perm)
        c = W2.shape[0]
        if c > 128:
            x2i = lax.bitcast_convert_type(x2[1].reshape(NPAD, c // 2, 2), jnp.int32)
            perm = _perm(c)
        else:
            x2i = x2[0]
            perm = None
        agg = _edge_agg(x2i, slab, counts, nchunk)
        xa, xb = x2[0], agg

    cl_pad = jnp.pad(cluster, (0, NPAD - N), constant_values=1 << 29)
    pooled_pad = _pool(xa, xb, cl_pad)
    invperm = np.argsort(_perm(512))
    pooled_fixed = jnp.concatenate(
        [pooled_pad[:, :512], pooled_pad[:, 512:][:, invperm]], axis=1)
    return _norm(pooled_fixed)


# speedup vs baseline: 1.0009x; 1.0009x over previous
import functools

import jax
import jax.numpy as jnp
from jax import lax
from jax.experimental import pallas as pl
from jax.experimental.pallas import tpu as pltpu
from jax.experimental.pallas import tpu_sc as plsc

N = 10000
NPAD = 10048
E = 320000
HIDDEN = 64
NCLUST = 1000
NW = 32
PAIR = 314
NR = 157
TILE_E = 2000
LOC_CAP = 4096
FLUSH = 2048
DUMMY = 511 * 16384
SLAB_W = E + FLUSH
ROW_BLK = 1256

_SC_PARAMS = pltpu.CompilerParams(needs_layout_passes=False)


def _mesh():
    return plsc.VectorSubcoreMesh(core_axis_name="c", subcore_axis_name="s")


def _wid():
    return lax.axis_index("s") * 2 + lax.axis_index("c")


def _mlp_body(xa_ref, xb_ref, w1a_ref, w1b_ref, b1_ref, g_ref, be_ref,
              w2_ref, b2_ref, o_ref, o2_ref):
    h = jnp.dot(xa_ref[...], w1a_ref[...], preferred_element_type=jnp.float32)
    h = h + jnp.dot(xb_ref[...], w1b_ref[...], preferred_element_type=jnp.float32)
    h = h + b1_ref[...]
    mu = jnp.mean(h, axis=-1, keepdims=True)
    var = jnp.mean((h - mu) * (h - mu), axis=-1, keepdims=True)
    h = (h - mu) * lax.rsqrt(var + 1e-5) * g_ref[...] + be_ref[...]
    h = jnp.maximum(h, 0.0)
    x2 = jnp.dot(h, w2_ref[...], preferred_element_type=jnp.float32) + b2_ref[...]
    o_ref[...] = x2
    o2_ref[...] = x2.astype(jnp.bfloat16)


def _mlp(xa, xb, W1, b1, g, be, W2, b2, perm):
    ca = xa.shape[1]
    if xb is None:
        xb = jnp.zeros((NPAD, 8), jnp.float32)
        w1b = jnp.zeros((8, HIDDEN), jnp.float32)
    else:
        w1b = W1[:, ca:].T
        if perm is not None:
            w1b = w1b[perm]
    w1a = W1[:, :ca].T
    cout = W2.shape[0]
    grid = NPAD // ROW_BLK
    return pl.pallas_call(
        _mlp_body,
        grid=(grid,),
        in_specs=[
            pl.BlockSpec((ROW_BLK, xa.shape[1]), lambda i: (i, 0)),
            pl.BlockSpec((ROW_BLK, xb.shape[1]), lambda i: (i, 0)),
            pl.BlockSpec((xa.shape[1], HIDDEN), lambda i: (0, 0)),
            pl.BlockSpec((xb.shape[1], HIDDEN), lambda i: (0, 0)),
            pl.BlockSpec((1, HIDDEN), lambda i: (0, 0)),
            pl.BlockSpec((1, HIDDEN), lambda i: (0, 0)),
            pl.BlockSpec((1, HIDDEN), lambda i: (0, 0)),
            pl.BlockSpec((HIDDEN, cout), lambda i: (0, 0)),
            pl.BlockSpec((1, cout), lambda i: (0, 0)),
        ],
        out_specs=[pl.BlockSpec((ROW_BLK, cout), lambda i: (i, 0)),
                   pl.BlockSpec((ROW_BLK, cout), lambda i: (i, 0))],
        out_shape=[jax.ShapeDtypeStruct((NPAD, cout), jnp.float32),
                   jax.ShapeDtypeStruct((NPAD, cout), jnp.bfloat16)],
    )(xa, xb, w1a, w1b, b1[None], g[None], be[None], W2.T, b2[None])


def _prep_body(src_hbm, dst_hbm, slab_hbm, counts_hbm, stile, dtile, loc, cntv):
    w = _wid()
    base = w * PAIR

    def init(i, _):
        loc[pl.ds(i * 16, 16)] = jnp.full((16,), DUMMY, jnp.int32)
        return 0
    lax.fori_loop(0, LOC_CAP // 16, init, 0)

    def tile_body(t, carry):
        off, written = carry
        pltpu.sync_copy(src_hbm.at[pl.ds(t * TILE_E, TILE_E)], stile)
        pltpu.sync_copy(dst_hbm.at[pl.ds(t * TILE_E, TILE_E)], dtile)

        def grp(g, off):
            d = dtile[pl.ds(g * 16, 16)]
            s = stile[pl.ds(g * 16, 16)]
            drel = d - base
            m = (drel >= 0) & (drel < PAIR)
            cs = plsc.cumsum(jnp.where(m, 1, 0))
            plsc.store_scatter(loc, [off + cs - 1], drel * 16384 + s, mask=m)
            return off + cs[15]
        off = lax.fori_loop(0, TILE_E // 16, grp, off)

        def do_flush(c):
            off, written = c
            pltpu.sync_copy(loc.at[pl.ds(0, FLUSH)],
                            slab_hbm.at[pl.ds(pl.multiple_of(w * SLAB_W + written, 2048), FLUSH)])

            def shift(i, _):
                loc[pl.ds(i * 16, 16)] = loc[pl.ds(FLUSH + i * 16, 16)]
                return 0
            lax.fori_loop(0, FLUSH // 16, shift, 0)
            return off - FLUSH, written + FLUSH
        off, written = lax.cond(off >= FLUSH, do_flush, lambda c: c, (off, written))
        return off, written

    off, written = lax.fori_loop(0, E // TILE_E, tile_body, (0, 0))

    def final_flush(c):
        off, written = c
        pltpu.sync_copy(loc.at[pl.ds(0, FLUSH)],
                        slab_hbm.at[pl.ds(pl.multiple_of(w * SLAB_W + written, 2048), FLUSH)])
        return 0, written + FLUSH
    off, written = lax.cond(off > 0, final_flush, lambda c: c, (off, written))

    cntv[pl.ds(0, 16)] = jnp.full((16,), written, jnp.int32)
    pltpu.sync_copy(cntv, counts_hbm.at[pl.ds(pl.multiple_of(w * 16, 16), 16)])


def _edge_prep(src, dst):
    return pl.kernel(
        _prep_body,
        out_type=(jax.ShapeDtypeStruct((NW * SLAB_W,), jnp.int32),
                  jax.ShapeDtypeStruct((NW * 16,), jnp.int32)),
        mesh=_mesh(),
        compiler_params=_SC_PARAMS,
        scratch_types=[
            pltpu.VMEM((TILE_E,), jnp.int32),
            pltpu.VMEM((TILE_E,), jnp.int32),
            pltpu.VMEM((LOC_CAP,), jnp.int32),
            pltpu.VMEM((16,), jnp.int32),
        ],
    )(src, dst)


def _agg_body(c, B, packed, x2i_hbm, slab_hbm, counts_hbm, agg_hbm,
              ltile, ldst, lsrc, idxb0, idxb1, gbuf0, gbuf1, acc, cntv,
              sem0, sem1):
    w = _wid()

    pltpu.sync_copy(counts_hbm.at[pl.ds(pl.multiple_of(w * 16, 16), 16)], cntv)
    cnt = cntv[pl.ds(0, 16)][0]
    ntiles = cnt // FLUSH

    def init_z(i, _):
        ldst[pl.ds(i * 16, 16)] = jnp.zeros((16,), jnp.int32)
        lsrc[pl.ds(i * 16, 16)] = jnp.zeros((16,), jnp.int32)
        return 0
    lax.fori_loop(0, (FLUSH + 144) // 16, init_z, 0)

    lane = lax.iota(jnp.int32, 16)

    for p in range(2):
        r = 2 * w + p
        rbase = p * NR

        def init_acc(i, _):
            acc[pl.ds(i * 16, 16)] = jnp.full((16,), -jnp.inf, jnp.float32)
            return 0
        lax.fori_loop(0, NR * c // 16, init_acc, 0)

        def tile_body(t, _):
            pltpu.sync_copy(slab_hbm.at[pl.ds(pl.multiple_of(w * SLAB_W + t * FLUSH, 2048), FLUSH)], ltile)

            def grp(g, off):
                pk = ltile[pl.ds(g * 16, 16)]
                dl = lax.shift_right_logical(pk, 14) - rbase
                s = pk & 16383
                m = (dl >= 0) & (dl < NR)
                cs = plsc.cumsum(jnp.where(m, 1, 0))
                idx = off + cs - 1
                plsc.store_scatter(ldst, [idx], dl, mask=m)
                plsc.store_scatter(lsrc, [idx], s, mask=m)
                return off + cs[15]
            off = lax.fori_loop(0, FLUSH // 16, grp, 0)

            base16 = (off // 16) * 16
            for k in range(8):
                gs = base16 + 16 * k
                v = ldst[pl.ds(gs, 16)]
                ldst[pl.ds(gs, 16)] = jnp.where(gs + lane >= off, NR, v)
            nb = (off + B - 1) // B

            def fire(b, idxb, gbuf, sem):
                for u in range(B // 16):
                    idxb[pl.ds(u * 16, 16)] = lsrc[pl.ds(b * B + u * 16, 16)]
                pltpu.async_copy(x2i_hbm.at[idxb], gbuf, sem)

            def wait(idxb, gbuf, sem):
                pltpu.make_async_copy(x2i_hbm.at[idxb], gbuf, sem).wait()

            def process(b, gbuf):
                def pgrp(gg, _):
                    dv = ldst[pl.ds(b * B + gg * 16, 16)]
                    for l in range(16):
                        rowb = dv[l] * c
                        e = gg * 16 + l
                        if packed:
                            nh = max(1, (c // 32) // 8)
                            bpn = (c // 32) // nh

                            def half(hh, _):
                                hb = hh * bpn
                                for u2 in range(bpn):
                                    wv = gbuf[e, pl.ds((hb + u2) * 16, 16)]
                                    lo = plsc.bitcast(lax.shift_left(wv, 16), jnp.float32)
                                    hi = plsc.bitcast(wv & (-65536), jnp.float32)
                                    cb = rowb + (hb + u2) * 32
                                    acc[pl.ds(cb, 16)] = jnp.maximum(acc[pl.ds(cb, 16)], lo)
                                    acc[pl.ds(cb + 16, 16)] = jnp.maximum(
                                        acc[pl.ds(cb + 16, 16)], hi)
                                return 0
                            lax.fori_loop(0, nh, half, 0)
                        else:
                            for u in range(c // 16):
                                acc[pl.ds(rowb + u * 16, 16)] = jnp.maximum(
                                    acc[pl.ds(rowb + u * 16, 16)],
                                    gbuf[e, pl.ds(u * 16, 16)])
                    return 0
                lax.fori_loop(0, B // 16, pgrp, 0)

            @pl.when(nb > 0)
            def _():
                fire(0, idxb0, gbuf0, sem0)

                def pair(b2, _):
                    b0 = 2 * b2
                    wait(idxb0, gbuf0, sem0)

                    @pl.when(b0 + 1 < nb)
                    def _():
                        fire(b0 + 1, idxb1, gbuf1, sem1)
                    process(b0, gbuf0)

                    @pl.when(b0 + 1 < nb)
                    def _():
                        wait(idxb1, gbuf1, sem1)

                        @pl.when(b0 + 2 < nb)
                        def _():
                            fire(b0 + 2, idxb0, gbuf0, sem0)
                        process(b0 + 1, gbuf1)
                    return 0
                lax.fori_loop(0, (nb + 1) // 2, pair, 0)
            return 0
        lax.fori_loop(0, ntiles, tile_body, 0)

        def fixup(i, _):
            v = acc[pl.ds(i * 16, 16)]
            acc[pl.ds(i * 16, 16)] = jnp.where(v < -1e37, 0.0, v)
            return 0
        lax.fori_loop(0, NR * c // 16, fixup, 0)
        pltpu.sync_copy(acc.at[pl.ds(0, NR * c)],
                        agg_hbm.at[pl.ds(pl.multiple_of(r * NR * c, 128), NR * c)])


def _edge_agg(x2i, slab, counts, nchunk):
    c = nchunk * 128
    packed = c > 128
    B = min(128, (32768 if packed else 16384) // c)
    wpr = c // 2 if packed else c
    out = pl.kernel(
        functools.partial(_agg_body, c, B, packed),
        out_type=jax.ShapeDtypeStruct((NPAD * c,), jnp.float32),
        mesh=_mesh(),
        compiler_params=_SC_PARAMS,
        scratch_types=[
            pltpu.VMEM((FLUSH,), jnp.int32),
            pltpu.VMEM((FLUSH + 144,), jnp.int32),
            pltpu.VMEM((FLUSH + 144,), jnp.int32),
            pltpu.VMEM((B,), jnp.int32),
            pltpu.VMEM((B,), jnp.int32),
            pltpu.VMEM((B, wpr), jnp.int32 if packed else jnp.float32),
            pltpu.VMEM((B, wpr), jnp.int32 if packed else jnp.float32),
            pltpu.VMEM(((NR + 1) * c,), jnp.float32),
            pltpu.VMEM((16,), jnp.int32),
            pltpu.SemaphoreType.DMA,
            pltpu.SemaphoreType.DMA,
        ],
    )(x2i, slab, counts)
    return out.reshape(NPAD, c)


CL_PER = 32
CTILE = 2512
RTILE = 64


def _pool_body(xa_hbm, xb_hbm, cl_hbm, out_hbm, ctile, crow, ra, rb, acc, sem):
    w = _wid()
    cbase = w * CL_PER

    def init_acc(i, _):
        for u in range(64):
            acc[i, pl.ds(u * 16, 16)] = jnp.full((16,), -jnp.inf, jnp.float32)
        return 0
    lax.fori_loop(0, CL_PER, init_acc, 0)

    def scan_tile(t, carry):
        lo, n = carry
        pltpu.sync_copy(cl_hbm.at[pl.ds(pl.multiple_of(t * CTILE, 16), CTILE)], ctile)

        def grp(g, carry):
            lo, n = carry
            v = ctile[pl.ds(g * 16, 16)]
            lo = lo + jnp.sum(jnp.where(v < cbase, 1, 0))
            n = n + jnp.sum(jnp.where((v >= cbase) & (v < cbase + CL_PER), 1, 0))
            return lo, n
        return lax.fori_loop(0, CTILE // 16, grp, carry)
    lo, nrows = lax.fori_loop(0, NPAD // CTILE, scan_tile, (0, 0))

    t0 = lo // RTILE
    t1 = (lo + nrows + RTILE - 1) // RTILE

    def tile_body(t, _):
        pltpu.sync_copy(xa_hbm.at[pl.ds(pl.multiple_of(t * RTILE, RTILE), RTILE), :], ra)
        pltpu.sync_copy(xb_hbm.at[pl.ds(pl.multiple_of(t * RTILE, RTILE), RTILE), :], rb)
        pltpu.sync_copy(cl_hbm.at[pl.ds(pl.multiple_of(t * RTILE, RTILE), RTILE)], crow.at[pl.ds(0, RTILE)])

        def row(e, _):
            d = crow[pl.ds(e, 16)][0] - cbase
            ok = (d >= 0) & (d < CL_PER)
            dc = jnp.clip(d, 0, CL_PER - 1)
            for u in range(32):
                va = jnp.where(ok, ra[e, pl.ds(u * 16, 16)], -jnp.inf)
                acc[dc, pl.ds(u * 16, 16)] = jnp.maximum(acc[dc, pl.ds(u * 16, 16)], va)
                vb = jnp.where(ok, rb[e, pl.ds(u * 16, 16)], -jnp.inf)
                acc[dc, pl.ds(512 + u * 16, 16)] = jnp.maximum(
                    acc[dc, pl.ds(512 + u * 16, 16)], vb)
            return 0
        lax.fori_loop(0, RTILE, row, 0)
        return 0
    lax.fori_loop(t0, t1, tile_body, 0)

    def fixup(i, _):
        for u in range(64):
            v = acc[i, pl.ds(u * 16, 16)]
            acc[i, pl.ds(u * 16, 16)] = jnp.where(v < -1e37, 0.0, v)
        return 0
    lax.fori_loop(0, CL_PER, fixup, 0)
    pltpu.sync_copy(acc, out_hbm.at[pl.ds(pl.multiple_of(w * CL_PER, CL_PER), CL_PER), :])


def _pool(xa, xb, cl_pad):
    return pl.kernel(
        _pool_body,
        out_type=jax.ShapeDtypeStruct((NW * CL_PER, 1024), jnp.float32),
        mesh=_mesh(),
        compiler_params=_SC_PARAMS,
        scratch_types=[
            pltpu.VMEM((CTILE,), jnp.int32),
            pltpu.VMEM((RTILE + 16,), jnp.int32),
            pltpu.VMEM((RTILE, 512), jnp.float32),
            pltpu.VMEM((RTILE, 512), jnp.float32),
            pltpu.VMEM((CL_PER, 1024), jnp.float32),
            pltpu.SemaphoreType.DMA,
        ],
    )(xa, xb, cl_pad)


def _norm_body(p_ref, o_ref):
    v = p_ref[...]
    ss = jnp.sum(v * v, axis=0, keepdims=True)
    o_ref[...] = v[:NCLUST, :] * lax.rsqrt(ss)


def _norm(pooled_pad):
    return pl.pallas_call(
        _norm_body,
        grid=(8,),
        in_specs=[pl.BlockSpec((NW * CL_PER, 128), lambda i: (0, i))],
        out_specs=pl.BlockSpec((NCLUST, 128), lambda i: (0, i)),
        out_shape=jax.ShapeDtypeStruct((NCLUST, 1024), jnp.float32),
    )(pooled_pad)


def _perm(c):
    import numpy as _np
    p = _np.arange(c).reshape(c // 32, 2, 16)
    return _np.concatenate([p[:, 0] * 1, p[:, 1]], axis=-1).reshape(-1) * 0 + _np.array(
        [b * 32 + 2 * k + h for b in range(c // 32) for h in range(2) for k in range(16)])


def kernel(x, edge_index, batch, cluster, W1_0, b1_0, g_0, be_0, W2_0, b2_0,
           W1_1, b1_1, g_1, be_1, W2_1, b2_1, W1_2, b1_2, g_2, be_2, W2_2, b2_2):
    import numpy as np
    src = edge_index[0]
    dst = edge_index[1]
    slab, counts = _edge_prep(src, dst)

    xa = jnp.pad(x, ((0, NPAD - N), (0, 0)))
    xb = None
    perm = None
    layers = [(W1_0, b1_0, g_0, be_0, W2_0, b2_0, 1),
              (W1_1, b1_1, g_1, be_1, W2_1, b2_1, 2),
              (W1_2, b1_2, g_2, be_2, W2_2, b2_2, 4)]
    for (W1, b1, g, be, W2, b2, nchunk) in layers:
        x2 = _mlp(xa, xb, W1, b1, g, be, W2, b2, perm)
        c = W2.shape[0]
        if c > 128:
            x2i = lax.bitcast_convert_type(x2[1].reshape(NPAD, c // 2, 2), jnp.int32)
            perm = _perm(c)
        else:
            x2i = x2[0]
            perm = None
        agg = _edge_agg(x2i, slab, counts, nchunk)
        xa, xb = x2[0], agg

    cl_pad = jnp.pad(cluster, (0, NPAD - N), constant_values=1 << 29)
    pooled_pad = _pool(xa, xb, cl_pad)
    invperm = np.argsort(_perm(512))
    pooled_fixed = jnp.concatenate(
        [pooled_pad[:, :512], pooled_pad[:, 512:][:, invperm]], axis=1)
    return _norm(pooled_fixed)


# speedup vs baseline: 1.3504x; 1.3491x over previous
import functools

import jax
import jax.numpy as jnp
from jax import lax
from jax.experimental import pallas as pl
from jax.experimental.pallas import tpu as pltpu
from jax.experimental.pallas import tpu_sc as plsc

N = 10000
NPAD = 10048
E = 320000
HIDDEN = 64
NCLUST = 1000
NW = 32
PAIR = 314
NR = 157
TILE_E = 2000
LOC_CAP = 4096
FLUSH = 2048
DUMMY = 511 * 16384
SLAB_W = E + FLUSH
ROW_BLK = 1256

_SC_PARAMS = pltpu.CompilerParams(needs_layout_passes=False)


def _mesh():
    return plsc.VectorSubcoreMesh(core_axis_name="c", subcore_axis_name="s")


def _wid():
    return lax.axis_index("s") * 2 + lax.axis_index("c")


def _mlp_body(xa_ref, xb_ref, w1a_ref, w1b_ref, b1_ref, g_ref, be_ref,
              w2_ref, b2_ref, o_ref, o2_ref):
    h = jnp.dot(xa_ref[...], w1a_ref[...], preferred_element_type=jnp.float32)
    h = h + jnp.dot(xb_ref[...], w1b_ref[...], preferred_element_type=jnp.float32)
    h = h + b1_ref[...]
    mu = jnp.mean(h, axis=-1, keepdims=True)
    var = jnp.mean((h - mu) * (h - mu), axis=-1, keepdims=True)
    h = (h - mu) * lax.rsqrt(var + 1e-5) * g_ref[...] + be_ref[...]
    h = jnp.maximum(h, 0.0)
    x2 = jnp.dot(h, w2_ref[...], preferred_element_type=jnp.float32) + b2_ref[...]
    o_ref[...] = x2
    o2_ref[...] = x2.astype(jnp.bfloat16)


def _mlp(xa, xb, W1, b1, g, be, W2, b2, perm):
    ca = xa.shape[1]
    if xb is None:
        xb = jnp.zeros((NPAD, 8), jnp.float32)
        w1b = jnp.zeros((8, HIDDEN), jnp.float32)
    else:
        w1b = W1[:, ca:].T
        if perm is not None:
            w1b = w1b[perm]
    w1a = W1[:, :ca].T
    cout = W2.shape[0]
    grid = NPAD // ROW_BLK
    return pl.pallas_call(
        _mlp_body,
        grid=(grid,),
        in_specs=[
            pl.BlockSpec((ROW_BLK, xa.shape[1]), lambda i: (i, 0)),
            pl.BlockSpec((ROW_BLK, xb.shape[1]), lambda i: (i, 0)),
            pl.BlockSpec((xa.shape[1], HIDDEN), lambda i: (0, 0)),
            pl.BlockSpec((xb.shape[1], HIDDEN), lambda i: (0, 0)),
            pl.BlockSpec((1, HIDDEN), lambda i: (0, 0)),
            pl.BlockSpec((1, HIDDEN), lambda i: (0, 0)),
            pl.BlockSpec((1, HIDDEN), lambda i: (0, 0)),
            pl.BlockSpec((HIDDEN, cout), lambda i: (0, 0)),
            pl.BlockSpec((1, cout), lambda i: (0, 0)),
        ],
        out_specs=[pl.BlockSpec((ROW_BLK, cout), lambda i: (i, 0)),
                   pl.BlockSpec((ROW_BLK, cout), lambda i: (i, 0))],
        out_shape=[jax.ShapeDtypeStruct((NPAD, cout), jnp.float32),
                   jax.ShapeDtypeStruct((NPAD, cout), jnp.bfloat16)],
    )(xa, xb, w1a, w1b, b1[None], g[None], be[None], W2.T, b2[None])


def _prep_body(src_hbm, dst_hbm, slab_hbm, counts_hbm, stile, dtile, loc, cntv):
    w = _wid()
    base = w * PAIR

    def init(i, _):
        loc[pl.ds(i * 16, 16)] = jnp.full((16,), DUMMY, jnp.int32)
        return 0
    lax.fori_loop(0, LOC_CAP // 16, init, 0)

    def tile_body(t, carry):
        off, written = carry
        pltpu.sync_copy(src_hbm.at[pl.ds(t * TILE_E, TILE_E)], stile)
        pltpu.sync_copy(dst_hbm.at[pl.ds(t * TILE_E, TILE_E)], dtile)

        def grp(g, off):
            d = dtile[pl.ds(g * 16, 16)]
            s = stile[pl.ds(g * 16, 16)]
            drel = d - base
            m = (drel >= 0) & (drel < PAIR)
            cs = plsc.cumsum(jnp.where(m, 1, 0))
            plsc.store_scatter(loc, [off + cs - 1], drel * 16384 + s, mask=m)
            return off + cs[15]
        off = lax.fori_loop(0, TILE_E // 16, grp, off)

        def do_flush(c):
            off, written = c
            pltpu.sync_copy(loc.at[pl.ds(0, FLUSH)],
                            slab_hbm.at[pl.ds(pl.multiple_of(w * SLAB_W + written, 2048), FLUSH)])

            def shift(i, _):
                loc[pl.ds(i * 16, 16)] = loc[pl.ds(FLUSH + i * 16, 16)]
                return 0
            lax.fori_loop(0, FLUSH // 16, shift, 0)
            return off - FLUSH, written + FLUSH
        off, written = lax.cond(off >= FLUSH, do_flush, lambda c: c, (off, written))
        return off, written

    off, written = lax.fori_loop(0, E // TILE_E, tile_body, (0, 0))

    def final_flush(c):
        off, written = c
        pltpu.sync_copy(loc.at[pl.ds(0, FLUSH)],
                        slab_hbm.at[pl.ds(pl.multiple_of(w * SLAB_W + written, 2048), FLUSH)])
        return 0, written + FLUSH
    off, written = lax.cond(off > 0, final_flush, lambda c: c, (off, written))

    cntv[pl.ds(0, 16)] = jnp.full((16,), written, jnp.int32)
    pltpu.sync_copy(cntv, counts_hbm.at[pl.ds(pl.multiple_of(w * 16, 16), 16)])


def _edge_prep(src, dst):
    return pl.kernel(
        _prep_body,
        out_type=(jax.ShapeDtypeStruct((NW * SLAB_W,), jnp.int32),
                  jax.ShapeDtypeStruct((NW * 16,), jnp.int32)),
        mesh=_mesh(),
        compiler_params=_SC_PARAMS,
        scratch_types=[
            pltpu.VMEM((TILE_E,), jnp.int32),
            pltpu.VMEM((TILE_E,), jnp.int32),
            pltpu.VMEM((LOC_CAP,), jnp.int32),
            pltpu.VMEM((16,), jnp.int32),
        ],
    )(src, dst)


def _agg_body(c, B, packed, nj, x2i_hbm, slab_hbm, counts_hbm, agg_hbm,
              ltile, ldst, lsrc, idxb0, idxb1, gbuf0, gbuf1, acc, cntv,
              sem0, sem1):
    w = _wid()

    pltpu.sync_copy(counts_hbm.at[pl.ds(pl.multiple_of(w * 16, 16), 16)], cntv)
    cnt = cntv[pl.ds(0, 16)][0]
    ntiles = cnt // FLUSH

    def init_z(i, _):
        ldst[pl.ds(i * 16, 16)] = jnp.zeros((16,), jnp.int32)
        lsrc[pl.ds(i * 16, 16)] = jnp.zeros((16,), jnp.int32)
        return 0
    lax.fori_loop(0, (FLUSH + 144) // 16, init_z, 0)

    lane = lax.iota(jnp.int32, 16)

    for p in range(2):
        r = 2 * w + p
        rbase = p * NR

        def init_acc(i, _):
            acc[pl.ds(i * 16, 16)] = jnp.full((16,), -jnp.inf, jnp.float32)
            return 0
        lax.fori_loop(0, NR * c // 16, init_acc, 0)

        def tile_body(t, _):
            pltpu.sync_copy(slab_hbm.at[pl.ds(pl.multiple_of(w * SLAB_W + t * FLUSH, 2048), FLUSH)], ltile)

            def grp(g, off):
                pk = ltile[pl.ds(g * 16, 16)]
                dl = lax.shift_right_logical(pk, 14) - rbase
                s = pk & 16383
                m = (dl >= 0) & (dl < NR)
                cs = plsc.cumsum(jnp.where(m, 1, 0))
                idx = off + cs - 1
                plsc.store_scatter(ldst, [idx], dl, mask=m)
                plsc.store_scatter(lsrc, [idx], s, mask=m)
                return off + cs[15]
            off = lax.fori_loop(0, FLUSH // 16, grp, 0)

            base16 = (off // 16) * 16
            for k in range(8):
                gs = base16 + 16 * k
                v = ldst[pl.ds(gs, 16)]
                ldst[pl.ds(gs, 16)] = jnp.where(gs + lane >= off, NR, v)
            nb = (off + B - 1) // B

            def fire(b, idxb, gbuf, sem, j):
                for u in range(B // 16):
                    sv = lsrc[pl.ds(b * B + u * 16, 16)]
                    idxb[pl.ds(u * 16, 16)] = sv * nj + j if nj > 1 else sv
                pltpu.async_copy(x2i_hbm.at[idxb], gbuf, sem)

            def wait(idxb, gbuf, sem):
                pltpu.make_async_copy(x2i_hbm.at[idxb], gbuf, sem).wait()

            def process(b, gbuf, j):
                cw = c // nj

                def pgrp(gg, _):
                    dv = ldst[pl.ds(b * B + gg * 16, 16)]
                    for l in range(16):
                        rowb = dv[l] * c + j * cw
                        e = gg * 16 + l
                        if packed:
                            for u2 in range(cw // 32):
                                wv = gbuf[e, pl.ds(u2 * 16, 16)]
                                lo = plsc.bitcast(lax.shift_left(wv, 16), jnp.float32)
                                hi = plsc.bitcast(wv & (-65536), jnp.float32)
                                cb = rowb + u2 * 32
                                acc[pl.ds(cb, 16)] = jnp.maximum(acc[pl.ds(cb, 16)], lo)
                                acc[pl.ds(cb + 16, 16)] = jnp.maximum(
                                    acc[pl.ds(cb + 16, 16)], hi)
                        else:
                            for u in range(cw // 16):
                                acc[pl.ds(rowb + u * 16, 16)] = jnp.maximum(
                                    acc[pl.ds(rowb + u * 16, 16)],
                                    gbuf[e, pl.ds(u * 16, 16)])
                    return 0
                lax.fori_loop(0, B // 16, pgrp, 0)

            for j in range(nj):
                @pl.when(nb > 0)
                def _():
                    fire(0, idxb0, gbuf0, sem0, j)

                    def pair(b2, _):
                        b0 = 2 * b2
                        wait(idxb0, gbuf0, sem0)

                        @pl.when(b0 + 1 < nb)
                        def _():
                            fire(b0 + 1, idxb1, gbuf1, sem1, j)
                        process(b0, gbuf0, j)

                        @pl.when(b0 + 1 < nb)
                        def _():
                            wait(idxb1, gbuf1, sem1)

                            @pl.when(b0 + 2 < nb)
                            def _():
                                fire(b0 + 2, idxb0, gbuf0, sem0, j)
                            process(b0 + 1, gbuf1, j)
                        return 0
                    lax.fori_loop(0, (nb + 1) // 2, pair, 0)
            return 0
        lax.fori_loop(0, ntiles, tile_body, 0)

        def fixup(i, _):
            v = acc[pl.ds(i * 16, 16)]
            acc[pl.ds(i * 16, 16)] = jnp.where(v < -1e37, 0.0, v)
            return 0
        lax.fori_loop(0, NR * c // 16, fixup, 0)
        pltpu.sync_copy(acc.at[pl.ds(0, NR * c)],
                        agg_hbm.at[pl.ds(pl.multiple_of(r * NR * c, 128), NR * c)])


def _edge_agg(x2i, slab, counts, nchunk):
    c = nchunk * 128
    packed = c > 128
    nj = max(1, c // 256) if packed else 1
    wpr = (c // nj) // 2 if packed else c
    if packed and nj > 1:
        x2i = x2i.reshape(NPAD * nj, wpr)
    B = 128
    out = pl.kernel(
        functools.partial(_agg_body, c, B, packed, nj),
        out_type=jax.ShapeDtypeStruct((NPAD * c,), jnp.float32),
        mesh=_mesh(),
        compiler_params=_SC_PARAMS,
        scratch_types=[
            pltpu.VMEM((FLUSH,), jnp.int32),
            pltpu.VMEM((FLUSH + 144,), jnp.int32),
            pltpu.VMEM((FLUSH + 144,), jnp.int32),
            pltpu.VMEM((B,), jnp.int32),
            pltpu.VMEM((B,), jnp.int32),
            pltpu.VMEM((B, wpr), jnp.int32 if packed else jnp.float32),
            pltpu.VMEM((B, wpr), jnp.int32 if packed else jnp.float32),
            pltpu.VMEM(((NR + 1) * c,), jnp.float32),
            pltpu.VMEM((16,), jnp.int32),
            pltpu.SemaphoreType.DMA,
            pltpu.SemaphoreType.DMA,
        ],
    )(x2i, slab, counts)
    return out.reshape(NPAD, c)


CL_PER = 32
CTILE = 2512
RTILE = 64


def _pool_body(xa_hbm, xb_hbm, cl_hbm, out_hbm, ctile, crow, ra, rb, acc, sem):
    w = _wid()
    cbase = w * CL_PER

    def init_acc(i, _):
        for u in range(64):
            acc[i, pl.ds(u * 16, 16)] = jnp.full((16,), -jnp.inf, jnp.float32)
        return 0
    lax.fori_loop(0, CL_PER, init_acc, 0)

    def scan_tile(t, carry):
        lo, n = carry
        pltpu.sync_copy(cl_hbm.at[pl.ds(pl.multiple_of(t * CTILE, 16), CTILE)], ctile)

        def grp(g, carry):
            lo, n = carry
            v = ctile[pl.ds(g * 16, 16)]
            lo = lo + jnp.sum(jnp.where(v < cbase, 1, 0))
            n = n + jnp.sum(jnp.where((v >= cbase) & (v < cbase + CL_PER), 1, 0))
            return lo, n
        return lax.fori_loop(0, CTILE // 16, grp, carry)
    lo, nrows = lax.fori_loop(0, NPAD // CTILE, scan_tile, (0, 0))

    t0 = lo // RTILE
    t1 = (lo + nrows + RTILE - 1) // RTILE

    def tile_body(t, _):
        pltpu.sync_copy(xa_hbm.at[pl.ds(pl.multiple_of(t * RTILE, RTILE), RTILE), :], ra)
        pltpu.sync_copy(xb_hbm.at[pl.ds(pl.multiple_of(t * RTILE, RTILE), RTILE), :], rb)
        pltpu.sync_copy(cl_hbm.at[pl.ds(pl.multiple_of(t * RTILE, RTILE), RTILE)], crow.at[pl.ds(0, RTILE)])

        def row(e, _):
            d = crow[pl.ds(e, 16)][0] - cbase
            ok = (d >= 0) & (d < CL_PER)
            dc = jnp.clip(d, 0, CL_PER - 1)
            for u in range(32):
                va = jnp.where(ok, ra[e, pl.ds(u * 16, 16)], -jnp.inf)
                acc[dc, pl.ds(u * 16, 16)] = jnp.maximum(acc[dc, pl.ds(u * 16, 16)], va)
                vb = jnp.where(ok, rb[e, pl.ds(u * 16, 16)], -jnp.inf)
                acc[dc, pl.ds(512 + u * 16, 16)] = jnp.maximum(
                    acc[dc, pl.ds(512 + u * 16, 16)], vb)
            return 0
        lax.fori_loop(0, RTILE, row, 0)
        return 0
    lax.fori_loop(t0, t1, tile_body, 0)

    def fixup(i, _):
        for u in range(64):
            v = acc[i, pl.ds(u * 16, 16)]
            acc[i, pl.ds(u * 16, 16)] = jnp.where(v < -1e37, 0.0, v)
        return 0
    lax.fori_loop(0, CL_PER, fixup, 0)
    pltpu.sync_copy(acc, out_hbm.at[pl.ds(pl.multiple_of(w * CL_PER, CL_PER), CL_PER), :])


def _pool(xa, xb, cl_pad):
    return pl.kernel(
        _pool_body,
        out_type=jax.ShapeDtypeStruct((NW * CL_PER, 1024), jnp.float32),
        mesh=_mesh(),
        compiler_params=_SC_PARAMS,
        scratch_types=[
            pltpu.VMEM((CTILE,), jnp.int32),
            pltpu.VMEM((RTILE + 16,), jnp.int32),
            pltpu.VMEM((RTILE, 512), jnp.float32),
            pltpu.VMEM((RTILE, 512), jnp.float32),
            pltpu.VMEM((CL_PER, 1024), jnp.float32),
            pltpu.SemaphoreType.DMA,
        ],
    )(xa, xb, cl_pad)


def _norm_body(p_ref, o_ref):
    v = p_ref[...]
    ss = jnp.sum(v * v, axis=0, keepdims=True)
    o_ref[...] = v[:NCLUST, :] * lax.rsqrt(ss)


def _norm(pooled_pad):
    return pl.pallas_call(
        _norm_body,
        grid=(8,),
        in_specs=[pl.BlockSpec((NW * CL_PER, 128), lambda i: (0, i))],
        out_specs=pl.BlockSpec((NCLUST, 128), lambda i: (0, i)),
        out_shape=jax.ShapeDtypeStruct((NCLUST, 1024), jnp.float32),
    )(pooled_pad)


def _perm(c):
    import numpy as _np
    p = _np.arange(c).reshape(c // 32, 2, 16)
    return _np.concatenate([p[:, 0] * 1, p[:, 1]], axis=-1).reshape(-1) * 0 + _np.array(
        [b * 32 + 2 * k + h for b in range(c // 32) for h in range(2) for k in range(16)])


def kernel(x, edge_index, batch, cluster, W1_0, b1_0, g_0, be_0, W2_0, b2_0,
           W1_1, b1_1, g_1, be_1, W2_1, b2_1, W1_2, b1_2, g_2, be_2, W2_2, b2_2):
    import numpy as np
    src = edge_index[0]
    dst = edge_index[1]
    slab, counts = _edge_prep(src, dst)

    xa = jnp.pad(x, ((0, NPAD - N), (0, 0)))
    xb = None
    perm = None
    layers = [(W1_0, b1_0, g_0, be_0, W2_0, b2_0, 1),
              (W1_1, b1_1, g_1, be_1, W2_1, b2_1, 2),
              (W1_2, b1_2, g_2, be_2, W2_2, b2_2, 4)]
    for (W1, b1, g, be, W2, b2, nchunk) in layers:
        x2 = _mlp(xa, xb, W1, b1, g, be, W2, b2, perm)
        c = W2.shape[0]
        if c > 128:
            x2i = lax.bitcast_convert_type(x2[1].reshape(NPAD, c // 2, 2), jnp.int32)
            perm = _perm(c)
        else:
            x2i = x2[0]
            perm = None
        agg = _edge_agg(x2i, slab, counts, nchunk)
        xa, xb = x2[0], agg

    cl_pad = jnp.pad(cluster, (0, NPAD - N), constant_values=1 << 29)
    pooled_pad = _pool(xa, xb, cl_pad)
    invperm = np.argsort(_perm(512))
    pooled_fixed = jnp.concatenate(
        [pooled_pad[:, :512], pooled_pad[:, 512:][:, invperm]], axis=1)
    return _norm(pooled_fixed)


# speedup vs baseline: 1.3518x; 1.0011x over previous
import functools

import jax
import jax.numpy as jnp
from jax import lax
from jax.experimental import pallas as pl
from jax.experimental.pallas import tpu as pltpu
from jax.experimental.pallas import tpu_sc as plsc

N = 10000
NPAD = 10048
E = 320000
HIDDEN = 64
NCLUST = 1000
NW = 32
PAIR = 314
NR = 157
TILE_E = 2000
LOC_CAP = 4096
FLUSH = 2048
DUMMY = 511 * 16384
SLAB_W = E + FLUSH
ROW_BLK = 1256

_SC_PARAMS = pltpu.CompilerParams(needs_layout_passes=False)


def _mesh():
    return plsc.VectorSubcoreMesh(core_axis_name="c", subcore_axis_name="s")


def _wid():
    return lax.axis_index("s") * 2 + lax.axis_index("c")


def _mlp_body(xa_ref, xb_ref, w1a_ref, w1b_ref, b1_ref, g_ref, be_ref,
              w2_ref, b2_ref, o_ref, o2_ref):
    h = jnp.dot(xa_ref[...], w1a_ref[...], preferred_element_type=jnp.float32)
    h = h + jnp.dot(xb_ref[...], w1b_ref[...], preferred_element_type=jnp.float32)
    h = h + b1_ref[...]
    mu = jnp.mean(h, axis=-1, keepdims=True)
    var = jnp.mean((h - mu) * (h - mu), axis=-1, keepdims=True)
    h = (h - mu) * lax.rsqrt(var + 1e-5) * g_ref[...] + be_ref[...]
    h = jnp.maximum(h, 0.0)
    x2 = jnp.dot(h, w2_ref[...], preferred_element_type=jnp.float32) + b2_ref[...]
    o_ref[...] = x2
    o2_ref[...] = x2.astype(jnp.bfloat16)


def _mlp(xa, xb, W1, b1, g, be, W2, b2, perm):
    ca = xa.shape[1]
    if xb is None:
        xb = jnp.zeros((NPAD, 8), jnp.float32)
        w1b = jnp.zeros((8, HIDDEN), jnp.float32)
    else:
        w1b = W1[:, ca:].T
        if perm is not None:
            w1b = w1b[perm]
    w1a = W1[:, :ca].T
    cout = W2.shape[0]
    grid = NPAD // ROW_BLK
    return pl.pallas_call(
        _mlp_body,
        grid=(grid,),
        in_specs=[
            pl.BlockSpec((ROW_BLK, xa.shape[1]), lambda i: (i, 0)),
            pl.BlockSpec((ROW_BLK, xb.shape[1]), lambda i: (i, 0)),
            pl.BlockSpec((xa.shape[1], HIDDEN), lambda i: (0, 0)),
            pl.BlockSpec((xb.shape[1], HIDDEN), lambda i: (0, 0)),
            pl.BlockSpec((1, HIDDEN), lambda i: (0, 0)),
            pl.BlockSpec((1, HIDDEN), lambda i: (0, 0)),
            pl.BlockSpec((1, HIDDEN), lambda i: (0, 0)),
            pl.BlockSpec((HIDDEN, cout), lambda i: (0, 0)),
            pl.BlockSpec((1, cout), lambda i: (0, 0)),
        ],
        out_specs=[pl.BlockSpec((ROW_BLK, cout), lambda i: (i, 0)),
                   pl.BlockSpec((ROW_BLK, cout), lambda i: (i, 0))],
        out_shape=[jax.ShapeDtypeStruct((NPAD, cout), jnp.float32),
                   jax.ShapeDtypeStruct((NPAD, cout), jnp.bfloat16)],
    )(xa, xb, w1a, w1b, b1[None], g[None], be[None], W2.T, b2[None])


def _prep_body(src_hbm, dst_hbm, slab_hbm, counts_hbm, stile, dtile, loc, cntv):
    w = _wid()
    base = w * PAIR

    def init(i, _):
        loc[pl.ds(i * 16, 16)] = jnp.full((16,), DUMMY, jnp.int32)
        return 0
    lax.fori_loop(0, LOC_CAP // 16, init, 0)

    def tile_body(t, carry):
        off, written = carry
        pltpu.sync_copy(src_hbm.at[pl.ds(t * TILE_E, TILE_E)], stile)
        pltpu.sync_copy(dst_hbm.at[pl.ds(t * TILE_E, TILE_E)], dtile)

        def grp(g, off):
            d = dtile[pl.ds(g * 16, 16)]
            s = stile[pl.ds(g * 16, 16)]
            drel = d - base
            m = (drel >= 0) & (drel < PAIR)
            cs = plsc.cumsum(jnp.where(m, 1, 0))
            plsc.store_scatter(loc, [off + cs - 1], drel * 16384 + s, mask=m)
            return off + cs[15]
        off = lax.fori_loop(0, TILE_E // 16, grp, off)

        def do_flush(c):
            off, written = c
            pltpu.sync_copy(loc.at[pl.ds(0, FLUSH)],
                            slab_hbm.at[pl.ds(pl.multiple_of(w * SLAB_W + written, 2048), FLUSH)])

            def shift(i, _):
                loc[pl.ds(i * 16, 16)] = loc[pl.ds(FLUSH + i * 16, 16)]
                return 0
            lax.fori_loop(0, FLUSH // 16, shift, 0)
            return off - FLUSH, written + FLUSH
        off, written = lax.cond(off >= FLUSH, do_flush, lambda c: c, (off, written))
        return off, written

    off, written = lax.fori_loop(0, E // TILE_E, tile_body, (0, 0))

    def final_flush(c):
        off, written = c
        pltpu.sync_copy(loc.at[pl.ds(0, FLUSH)],
                        slab_hbm.at[pl.ds(pl.multiple_of(w * SLAB_W + written, 2048), FLUSH)])
        return 0, written + FLUSH
    off, written = lax.cond(off > 0, final_flush, lambda c: c, (off, written))

    cntv[pl.ds(0, 16)] = jnp.full((16,), written, jnp.int32)
    pltpu.sync_copy(cntv, counts_hbm.at[pl.ds(pl.multiple_of(w * 16, 16), 16)])


def _edge_prep(src, dst):
    return pl.kernel(
        _prep_body,
        out_type=(jax.ShapeDtypeStruct((NW * SLAB_W,), jnp.int32),
                  jax.ShapeDtypeStruct((NW * 16,), jnp.int32)),
        mesh=_mesh(),
        compiler_params=_SC_PARAMS,
        scratch_types=[
            pltpu.VMEM((TILE_E,), jnp.int32),
            pltpu.VMEM((TILE_E,), jnp.int32),
            pltpu.VMEM((LOC_CAP,), jnp.int32),
            pltpu.VMEM((16,), jnp.int32),
        ],
    )(src, dst)


def _agg_body(c, B, packed, nj, x2i_hbm, slab_hbm, counts_hbm, agg_hbm,
              ltile, ldst, lsrc, idxb0, idxb1, gbuf0, gbuf1, acc, cntv,
              sem0, sem1):
    w = _wid()

    pltpu.sync_copy(counts_hbm.at[pl.ds(pl.multiple_of(w * 16, 16), 16)], cntv)
    cnt = cntv[pl.ds(0, 16)][0]
    ntiles = cnt // FLUSH

    def init_z(i, _):
        ldst[pl.ds(i * 16, 16)] = jnp.zeros((16,), jnp.int32)
        lsrc[pl.ds(i * 16, 16)] = jnp.zeros((16,), jnp.int32)
        return 0
    lax.fori_loop(0, (FLUSH + 144) // 16, init_z, 0)

    lane = lax.iota(jnp.int32, 16)

    for p in range(2):
        r = 2 * w + p
        rbase = p * NR

        def init_acc(i, _):
            acc[pl.ds(i * 16, 16)] = jnp.full((16,), -jnp.inf, jnp.float32)
            return 0
        lax.fori_loop(0, NR * c // 16, init_acc, 0)

        def tile_body(t, _):
            pltpu.sync_copy(slab_hbm.at[pl.ds(pl.multiple_of(w * SLAB_W + t * FLUSH, 2048), FLUSH)], ltile)

            def grp(g, off):
                pk = ltile[pl.ds(g * 16, 16)]
                dl = lax.shift_right_logical(pk, 14) - rbase
                s = pk & 16383
                m = (dl >= 0) & (dl < NR)
                cs = plsc.cumsum(jnp.where(m, 1, 0))
                idx = off + cs - 1
                plsc.store_scatter(ldst, [idx], dl, mask=m)
                plsc.store_scatter(lsrc, [idx], s, mask=m)
                return off + cs[15]
            off = lax.fori_loop(0, FLUSH // 16, grp, 0)

            base16 = (off // 16) * 16
            for k in range(8):
                gs = base16 + 16 * k
                v = ldst[pl.ds(gs, 16)]
                ldst[pl.ds(gs, 16)] = jnp.where(gs + lane >= off, NR, v)
            nb = (off + B - 1) // B

            def fire(b, idxb, gbuf, sem, j):
                for u in range(B // 16):
                    sv = lsrc[pl.ds(b * B + u * 16, 16)]
                    idxb[pl.ds(u * 16, 16)] = sv * nj + j if nj > 1 else sv
                pltpu.async_copy(x2i_hbm.at[idxb], gbuf, sem)

            def wait(idxb, gbuf, sem):
                pltpu.make_async_copy(x2i_hbm.at[idxb], gbuf, sem).wait()

            def process(b, gbuf, j):
                cw = c // nj

                def pgrp(gg, _):
                    dv = ldst[pl.ds(b * B + gg * 16, 16)]
                    for l in range(16):
                        rowb = dv[l] * c + j * cw
                        e = gg * 16 + l
                        if packed:
                            for u2 in range(cw // 32):
                                wv = gbuf[e, pl.ds(u2 * 16, 16)]
                                lo = plsc.bitcast(lax.shift_left(wv, 16), jnp.float32)
                                hi = plsc.bitcast(wv & (-65536), jnp.float32)
                                cb = rowb + u2 * 32
                                acc[pl.ds(cb, 16)] = jnp.maximum(acc[pl.ds(cb, 16)], lo)
                                acc[pl.ds(cb + 16, 16)] = jnp.maximum(
                                    acc[pl.ds(cb + 16, 16)], hi)
                        else:
                            for u in range(cw // 16):
                                acc[pl.ds(rowb + u * 16, 16)] = jnp.maximum(
                                    acc[pl.ds(rowb + u * 16, 16)],
                                    gbuf[e, pl.ds(u * 16, 16)])
                    return 0
                lax.fori_loop(0, B // 16, pgrp, 0)

            for j in range(nj):
                @pl.when(nb > 0)
                def _():
                    fire(0, idxb0, gbuf0, sem0, j)

                    def pair(b2, _):
                        b0 = 2 * b2
                        wait(idxb0, gbuf0, sem0)

                        @pl.when(b0 + 1 < nb)
                        def _():
                            fire(b0 + 1, idxb1, gbuf1, sem1, j)
                        process(b0, gbuf0, j)

                        @pl.when(b0 + 1 < nb)
                        def _():
                            wait(idxb1, gbuf1, sem1)

                            @pl.when(b0 + 2 < nb)
                            def _():
                                fire(b0 + 2, idxb0, gbuf0, sem0, j)
                            process(b0 + 1, gbuf1, j)
                        return 0
                    lax.fori_loop(0, (nb + 1) // 2, pair, 0)
            return 0
        lax.fori_loop(0, ntiles, tile_body, 0)

        def fixup(i, _):
            v = acc[pl.ds(i * 16, 16)]
            acc[pl.ds(i * 16, 16)] = jnp.where(v < -1e37, 0.0, v)
            return 0
        lax.fori_loop(0, NR * c // 16, fixup, 0)
        pltpu.sync_copy(acc.at[pl.ds(0, NR * c)],
                        agg_hbm.at[pl.ds(pl.multiple_of(r * NR * c, 128), NR * c)])


def _edge_agg(x2i, slab, counts, nchunk):
    c = nchunk * 128
    packed = c > 128
    nj = max(1, c // 256) if packed else 1
    wpr = (c // nj) // 2 if packed else c
    if packed and nj > 1:
        x2i = x2i.reshape(NPAD * nj, wpr)
    B = 128
    out = pl.kernel(
        functools.partial(_agg_body, c, B, packed, nj),
        out_type=jax.ShapeDtypeStruct((NPAD * c,), jnp.float32),
        mesh=_mesh(),
        compiler_params=_SC_PARAMS,
        scratch_types=[
            pltpu.VMEM((FLUSH,), jnp.int32),
            pltpu.VMEM((FLUSH + 144,), jnp.int32),
            pltpu.VMEM((FLUSH + 144,), jnp.int32),
            pltpu.VMEM((B,), jnp.int32),
            pltpu.VMEM((B,), jnp.int32),
            pltpu.VMEM((B, wpr), jnp.int32 if packed else jnp.float32),
            pltpu.VMEM((B, wpr), jnp.int32 if packed else jnp.float32),
            pltpu.VMEM(((NR + 1) * c,), jnp.float32),
            pltpu.VMEM((16,), jnp.int32),
            pltpu.SemaphoreType.DMA,
            pltpu.SemaphoreType.DMA,
        ],
    )(x2i, slab, counts)
    return out.reshape(NPAD, c)


CL_PER = 32
CTILE = 2512
RTILE = 64


def _pool_body(xa_hbm, xb_hbm, cl_hbm, out_hbm, ctile, crow, ra, rb, acc, sem):
    w = _wid()
    cbase = w * CL_PER

    def init_acc(i, _):
        for u in range(64):
            acc[i, pl.ds(u * 16, 16)] = jnp.full((16,), -jnp.inf, jnp.float32)
        return 0
    lax.fori_loop(0, CL_PER, init_acc, 0)

    def scan_tile(t, carry):
        lo, n = carry
        pltpu.sync_copy(cl_hbm.at[pl.ds(pl.multiple_of(t * CTILE, 16), CTILE)], ctile)

        def grp(g, carry):
            lo, n = carry
            v = ctile[pl.ds(g * 16, 16)]
            lo = lo + jnp.sum(jnp.where(v < cbase, 1, 0))
            n = n + jnp.sum(jnp.where((v >= cbase) & (v < cbase + CL_PER), 1, 0))
            return lo, n
        return lax.fori_loop(0, CTILE // 16, grp, carry)
    lo, nrows = lax.fori_loop(0, NPAD // CTILE, scan_tile, (0, 0))

    t0 = lo // RTILE
    t1 = (lo + nrows + RTILE - 1) // RTILE

    def tile_body(t, _):
        pltpu.sync_copy(xa_hbm.at[pl.ds(pl.multiple_of(t * RTILE, RTILE), RTILE), :], ra)
        pltpu.sync_copy(xb_hbm.at[pl.ds(pl.multiple_of(t * RTILE, RTILE), RTILE), :], rb)
        pltpu.sync_copy(cl_hbm.at[pl.ds(pl.multiple_of(t * RTILE, RTILE), RTILE)], crow.at[pl.ds(0, RTILE)])

        def row(e, _):
            d = crow[pl.ds(e, 16)][0] - cbase
            ok = (d >= 0) & (d < CL_PER)
            dc = jnp.clip(d, 0, CL_PER - 1)
            for u in range(32):
                va = jnp.where(ok, ra[e, pl.ds(u * 16, 16)], -jnp.inf)
                acc[dc, pl.ds(u * 16, 16)] = jnp.maximum(acc[dc, pl.ds(u * 16, 16)], va)
                vb = jnp.where(ok, rb[e, pl.ds(u * 16, 16)], -jnp.inf)
                acc[dc, pl.ds(512 + u * 16, 16)] = jnp.maximum(
                    acc[dc, pl.ds(512 + u * 16, 16)], vb)
            return 0
        lax.fori_loop(0, RTILE, row, 0)
        return 0
    lax.fori_loop(t0, t1, tile_body, 0)

    def fixup(i, _):
        for u in range(64):
            v = acc[i, pl.ds(u * 16, 16)]
            acc[i, pl.ds(u * 16, 16)] = jnp.where(v < -1e37, 0.0, v)
        return 0
    lax.fori_loop(0, CL_PER, fixup, 0)
    pltpu.sync_copy(acc, out_hbm.at[pl.ds(pl.multiple_of(w * CL_PER, CL_PER), CL_PER), :])


def _pool(xa, xb, cl_pad):
    return pl.kernel(
        _pool_body,
        out_type=jax.ShapeDtypeStruct((NW * CL_PER, 1024), jnp.float32),
        mesh=_mesh(),
        compiler_params=_SC_PARAMS,
        scratch_types=[
            pltpu.VMEM((CTILE,), jnp.int32),
            pltpu.VMEM((RTILE + 16,), jnp.int32),
            pltpu.VMEM((RTILE, 512), jnp.float32),
            pltpu.VMEM((RTILE, 512), jnp.float32),
            pltpu.VMEM((CL_PER, 1024), jnp.float32),
            pltpu.SemaphoreType.DMA,
        ],
    )(xa, xb, cl_pad)


def _norm_body(p_ref, o_ref):
    v = p_ref[...]
    ss = jnp.sum(v * v, axis=0, keepdims=True)
    o_ref[...] = v[:NCLUST, :] * lax.rsqrt(ss)


def _norm(pooled_pad):
    return pl.pallas_call(
        _norm_body,
        grid=(8,),
        in_specs=[pl.BlockSpec((NW * CL_PER, 128), lambda i: (0, i))],
        out_specs=pl.BlockSpec((NCLUST, 128), lambda i: (0, i)),
        out_shape=jax.ShapeDtypeStruct((NCLUST, 1024), jnp.float32),
    )(pooled_pad)


def _perm(c):
    import numpy as _np
    return _np.array([b * 32 + 2 * k + h
                      for b in range(c // 32) for h in range(2) for k in range(16)])


def kernel(x, edge_index, batch, cluster, W1_0, b1_0, g_0, be_0, W2_0, b2_0,
           W1_1, b1_1, g_1, be_1, W2_1, b2_1, W1_2, b1_2, g_2, be_2, W2_2, b2_2):
    import numpy as np
    src = edge_index[0]
    dst = edge_index[1]
    slab, counts = _edge_prep(src, dst)

    xa = jnp.pad(x, ((0, NPAD - N), (0, 0)))
    xb = None
    perm = None
    layers = [(W1_0, b1_0, g_0, be_0, W2_0, b2_0, 1),
              (W1_1, b1_1, g_1, be_1, W2_1, b2_1, 2),
              (W1_2, b1_2, g_2, be_2, W2_2, b2_2, 4)]
    for (W1, b1, g, be, W2, b2, nchunk) in layers:
        x2 = _mlp(xa, xb, W1, b1, g, be, W2, b2, perm)
        c = W2.shape[0]
        if c > 128:
            x2i = lax.bitcast_convert_type(x2[1].reshape(NPAD, c // 2, 2), jnp.int32)
            perm = _perm(c)
        else:
            x2i = x2[0]
            perm = None
        agg = _edge_agg(x2i, slab, counts, nchunk)
        xa, xb = x2[0], agg

    cl_pad = jnp.pad(cluster, (0, NPAD - N), constant_values=1 << 29)
    pooled_pad = _pool(xa, xb, cl_pad)
    invperm = np.argsort(_perm(512))
    pooled_fixed = jnp.concatenate(
        [pooled_pad[:, :512], pooled_pad[:, 512:][:, invperm]], axis=1)
    return _norm(pooled_fixed)
